# Initial kernel scaffold; baseline (speedup 1.0000x reference)
import jax
import jax.numpy as jnp
from jax.experimental import pallas as pl


def kernel(x, w_cv1, g_cv1, b_cv1, w_cv2, g_cv2, b_cv2, w_m1, g_m1, b_m1, w_m2, g_m2, b_m2, w_cv3, g_cv3, b_cv3):
    raise NotImplementedError("write your pallas kernel here")


# speedup vs baseline: 147.1829x; 147.1829x over previous
import functools
import jax
import jax.numpy as jnp
from jax.experimental import pallas as pl
from jax.experimental.pallas import tpu as pltpu

P = 3
L = 8
_DIRS = [(0, 1), (1, 0), (1, 1), (1, -1)]

_PAIRS = []
for _dr, _dc in _DIRS:
    for _i in range(P):
        for _j in range(P):
            if 0 <= _i + _dr < P and 0 <= _j + _dc < P:
                _PAIRS.append((_i * 3 + _j, (_i + _dr) * 3 + (_j + _dc)))

_WPAD = 58
_TLEN = 55 * 58 + 56
_FLEN = 56 * 58
_NPIX = 58 * 58


def _glcm_kernel(xp_ref, out_ref):
    q = jnp.clip(xp_ref[...] * (L - 1), 0, L - 1).astype(jnp.int32)
    nbr = [q[:, (di * _WPAD + dj):(di * _WPAD + dj) + _TLEN]
           for di in range(P) for dj in range(P)]
    s8 = [n * L for n in nbr]
    bins = [s8[a] + nbr[b] for a, b in _PAIRS]
    d = [nbr[a] - nbr[b] for a, b in _PAIRS]

    contrast = d[0] * d[0]
    for dk in d[1:]:
        contrast = contrast + dk * dk

    habs = [jnp.abs(dk).astype(jnp.float32) for dk in d]
    homog = 1.0 / (1.0 + habs[0])
    for hk in habs[1:]:
        homog = homog + 1.0 / (1.0 + hk)

    one = jnp.ones_like(bins[0])
    zero = jnp.zeros_like(bins[0])
    m = [one] * 20
    for k in range(20):
        for l in range(k + 1, 20):
            e = jnp.where(bins[k] == bins[l], one, zero)
            m[k] = m[k] + e
            m[l] = m[l] + e

    energy = m[0]
    for mk in m[1:]:
        energy = energy + mk

    inv20 = 1.0 / 20.0
    prod = m[0].astype(jnp.float32) * inv20 + 1e-6
    for mk in m[1:]:
        prod = prod * (mk.astype(jnp.float32) * inv20 + 1e-6)
    logsum = jnp.log(prod)

    fb = jnp.stack([
        jnp.sum(contrast.astype(jnp.float32), axis=0),
        jnp.sum(energy.astype(jnp.float32), axis=0),
        jnp.sum(logsum, axis=0),
        jnp.sum(homog, axis=0),
    ], axis=0)

    @pl.when(pl.program_id(0) == 0)
    def _():
        out_ref[...] = jnp.zeros_like(out_ref)

    out_ref[...] += fb


def _conv_kernel(x_ref, f4_ref,
                 w1x_ref, w1f_ref, g1_ref, b1_ref,
                 w2x_ref, w2f_ref, g2_ref, b2_ref,
                 wm1_ref, gm1_ref, bm1_ref,
                 wm2_ref, gm2_ref, bm2_ref,
                 w3a_ref, w3b_ref, g3_ref, b3_ref,
                 out_ref, pad_ref):
    nch = x_ref.shape[0]

    def bnsilu(y, g_ref, b_ref):
        y = y * (g_ref[...] * (1.0 / jnp.sqrt(1.0 + 0.001))) + b_ref[...]
        return y * jax.nn.sigmoid(y)

    def mm(w_ref, a):
        return jax.lax.dot_general(w_ref[...], a, (((1,), (0,)), ((), ())),
                                   preferred_element_type=jnp.float32)

    x = x_ref[...]
    f4t = f4_ref[...]
    cN = 1.0 / (20.0 * nch)
    rid = jax.lax.broadcasted_iota(jnp.int32, (4, 1), 0)
    scale = jnp.where(rid == 1, 1.0 / (400.0 * nch),
                      jnp.where(rid == 2, -cN, cN))
    f4t = f4t * scale
    f4 = jnp.pad(f4t, ((0, 0), (0, _FLEN - _TLEN)))

    y1 = mm(w1x_ref, x) + mm(w1f_ref, f4)
    h1 = bnsilu(y1, g1_ref, b1_ref)
    m1 = bnsilu(mm(wm1_ref, h1), gm1_ref, bm1_ref)

    lane = jax.lax.broadcasted_iota(jnp.int32, (1, _FLEN), 1)
    m1 = jnp.where((lane % _WPAD) < 56, m1, 0.0)

    pad_ref[...] = jnp.zeros_like(pad_ref)
    pad_ref[:, (_WPAD + 1):(_WPAD + 1) + _FLEN] = m1
    y2 = None
    for ki in range(3):
        for kj in range(3):
            o = ki * _WPAD + kj
            w = wm2_ref[:, :, ki, kj]
            t = jax.lax.dot_general(w, pad_ref[:, o:o + _FLEN],
                                    (((1,), (0,)), ((), ())),
                                    preferred_element_type=jnp.float32)
            y2 = t if y2 is None else y2 + t
    m2 = bnsilu(y2, gm2_ref, bm2_ref)

    mo = h1 + m2
    h2 = bnsilu(mm(w2x_ref, x) + mm(w2f_ref, f4), g2_ref, b2_ref)
    y3 = mm(w3a_ref, mo) + mm(w3b_ref, h2)
    out_ref[...] = bnsilu(y3, g3_ref, b3_ref)


def kernel(x, w_cv1, g_cv1, b_cv1, w_cv2, g_cv2, b_cv2, w_m1, g_m1, b_m1,
           w_m2, g_m2, b_m2, w_cv3, g_cv3, b_cv3):
    B, C, H, W = x.shape
    assert (B, H, W) == (1, 56, 56)
    x0 = x[0]

    xp = jnp.pad(x0, ((0, 0), (1, 1), (1, 1)), mode='edge')
    xp = xp.reshape(C, _NPIX)
    CB = 8
    f4_raw = pl.pallas_call(
        _glcm_kernel,
        grid=(C // CB,),
        in_specs=[pl.BlockSpec((CB, _NPIX), lambda i: (i, 0))],
        out_specs=pl.BlockSpec((4, _TLEN), lambda i: (0, 0)),
        out_shape=jax.ShapeDtypeStruct((4, _TLEN), jnp.float32),
        compiler_params=pltpu.CompilerParams(
            dimension_semantics=("arbitrary",)),
    )(xp)

    xw = jnp.pad(x0, ((0, 0), (0, 0), (0, 2))).reshape(C, _FLEN)
    c_ = w_cv1.shape[0]
    ch = w_m1.shape[0]
    c2 = w_cv3.shape[0]
    w1 = w_cv1[:, :, 0, 0]
    w2 = w_cv2[:, :, 0, 0]
    w3 = w_cv3[:, :, 0, 0]
    col = lambda v: v[:, None]

    out_flat = pl.pallas_call(
        _conv_kernel,
        out_shape=jax.ShapeDtypeStruct((c2, _FLEN), jnp.float32),
        scratch_shapes=[pltpu.VMEM((ch, _NPIX + 2), jnp.float32)],
    )(xw, f4_raw,
      w1[:, :C], w1[:, C:], col(g_cv1), col(b_cv1),
      w2[:, :C], w2[:, C:], col(g_cv2), col(b_cv2),
      w_m1[:, :, 0, 0], col(g_m1), col(b_m1),
      w_m2, col(g_m2), col(b_m2),
      w3[:, :c_], w3[:, c_:], col(g_cv3), col(b_cv3))

    return out_flat.reshape(c2, 56, _WPAD)[None, :, :, :56]


# speedup vs baseline: 245.5563x; 1.6684x over previous
import functools
import jax
import jax.numpy as jnp
from jax.experimental import pallas as pl
from jax.experimental.pallas import tpu as pltpu

P = 3
L = 8
_DIRS = [(0, 1), (1, 0), (1, 1), (1, -1)]

_PAIRS = []
for _dr, _dc in _DIRS:
    for _i in range(P):
        for _j in range(P):
            if 0 <= _i + _dr < P and 0 <= _j + _dc < P:
                _PAIRS.append((_i * 3 + _j, (_i + _dr) * 3 + (_j + _dc)))

_WPAD = 58
_FLEN = 56 * _WPAD
_XLEN = 3368


def _glcm_kernel(xp_ref, out_ref, nb_ref):
    cb = xp_ref.shape[0]
    q = jnp.clip(xp_ref[...] * (L - 1), 0, L - 1).astype(jnp.int32)
    for i in range(9):
        o = (i // 3) * _WPAD + (i % 3)
        nb_ref[i * cb:(i + 1) * cb, :] = q[:, o:o + _FLEN]
    nbr = [nb_ref[i * cb:(i + 1) * cb, :] for i in range(9)]
    s8 = [n * L for n in nbr]
    bins = [s8[a] + nbr[b] for a, b in _PAIRS]
    d = [nbr[a] - nbr[b] for a, b in _PAIRS]

    contrast = d[0] * d[0]
    for dk in d[1:]:
        contrast = contrast + dk * dk

    habs = [jnp.abs(dk).astype(jnp.float32) for dk in d]
    homog = 1.0 / (1.0 + habs[0])
    for hk in habs[1:]:
        homog = homog + 1.0 / (1.0 + hk)

    one = jnp.ones_like(bins[0])
    zero = jnp.zeros_like(bins[0])
    inv20 = 1.0 / 20.0
    energy = None
    prod = None
    for k in range(20):
        mk = one
        for l in range(20):
            if l == k:
                continue
            mk = mk + jnp.where(bins[k] == bins[l], one, zero)
        energy = mk if energy is None else energy + mk
        p = mk.astype(jnp.float32) * inv20 + 1e-6
        prod = p if prod is None else prod * p
    logsum = jnp.log(prod)

    fb = jnp.stack([
        jnp.sum(contrast.astype(jnp.float32), axis=0),
        jnp.sum(energy.astype(jnp.float32), axis=0),
        jnp.sum(logsum, axis=0),
        jnp.sum(homog, axis=0),
    ], axis=0)

    @pl.when(pl.program_id(0) == 0)
    def _():
        out_ref[...] = jnp.zeros_like(out_ref)

    out_ref[...] += fb


def _conv_kernel(x_ref, f4_ref,
                 w1x_ref, w1f_ref, g1_ref, b1_ref,
                 w2x_ref, w2f_ref, g2_ref, b2_ref,
                 wm1_ref, gm1_ref, bm1_ref,
                 wm2_ref, gm2_ref, bm2_ref,
                 w3a_ref, w3b_ref, g3_ref, b3_ref,
                 out_ref, pad_ref, col_ref):
    nch = x_ref.shape[0]

    def bnsilu(y, g_ref, b_ref):
        y = y * (g_ref[...] * (1.0 / jnp.sqrt(1.0 + 0.001))) + b_ref[...]
        return y * jax.nn.sigmoid(y)

    def mm(w_ref, a):
        return jax.lax.dot_general(w_ref[...], a, (((1,), (0,)), ((), ())),
                                   preferred_element_type=jnp.float32)

    x = x_ref[...]
    cN = 1.0 / (20.0 * nch)
    rid = jax.lax.broadcasted_iota(jnp.int32, (4, 1), 0)
    scale = jnp.where(rid == 1, 1.0 / (400.0 * nch),
                      jnp.where(rid == 2, -cN, cN))
    f4 = f4_ref[...] * scale

    y1 = mm(w1x_ref, x) + mm(w1f_ref, f4)
    h1 = bnsilu(y1, g1_ref, b1_ref)
    m1 = bnsilu(mm(wm1_ref, h1), gm1_ref, bm1_ref)

    lane = jax.lax.broadcasted_iota(jnp.int32, (1, _FLEN), 1)
    m1 = jnp.where((lane % _WPAD) < 56, m1, 0.0)

    nm = wm2_ref.shape[1] // 9
    pad_ref[...] = jnp.zeros_like(pad_ref)
    pad_ref[:, (_WPAD + 1):(_WPAD + 1) + _FLEN] = m1
    for g in range(9):
        o = (g // 3) * _WPAD + (g % 3)
        col_ref[g * nm:(g + 1) * nm, :] = pad_ref[:, o:o + _FLEN]
    y2 = mm(wm2_ref, col_ref[...])
    m2 = bnsilu(y2, gm2_ref, bm2_ref)

    mo = h1 + m2
    h2 = bnsilu(mm(w2x_ref, x) + mm(w2f_ref, f4), g2_ref, b2_ref)
    y3 = mm(w3a_ref, mo) + mm(w3b_ref, h2)
    out_ref[...] = bnsilu(y3, g3_ref, b3_ref)


def kernel(x, w_cv1, g_cv1, b_cv1, w_cv2, g_cv2, b_cv2, w_m1, g_m1, b_m1,
           w_m2, g_m2, b_m2, w_cv3, g_cv3, b_cv3):
    B, C, H, W = x.shape
    assert (B, H, W) == (1, 56, 56)
    x0 = x[0]

    xp = jnp.pad(x0, ((0, 0), (1, 1), (1, 1)), mode='edge')
    xp = jnp.pad(xp.reshape(C, _WPAD * _WPAD),
                 ((0, 0), (0, _XLEN - _WPAD * _WPAD)))
    CB = 8
    f4_raw = pl.pallas_call(
        _glcm_kernel,
        grid=(C // CB,),
        in_specs=[pl.BlockSpec((CB, _XLEN), lambda i: (i, 0))],
        out_specs=pl.BlockSpec((4, _FLEN), lambda i: (0, 0)),
        out_shape=jax.ShapeDtypeStruct((4, _FLEN), jnp.float32),
        scratch_shapes=[pltpu.VMEM((9 * CB, _FLEN), jnp.int32)],
        compiler_params=pltpu.CompilerParams(
            dimension_semantics=("arbitrary",)),
    )(xp)

    xw = jnp.pad(x0, ((0, 0), (0, 0), (0, 2))).reshape(C, _FLEN)
    c_ = w_cv1.shape[0]
    ch = w_m1.shape[0]
    c2 = w_cv3.shape[0]
    w1 = w_cv1[:, :, 0, 0]
    w2 = w_cv2[:, :, 0, 0]
    w3 = w_cv3[:, :, 0, 0]
    wm2s = w_m2.transpose(0, 2, 3, 1).reshape(c_, 9 * ch)
    col = lambda v: v[:, None]

    out_flat = pl.pallas_call(
        _conv_kernel,
        out_shape=jax.ShapeDtypeStruct((c2, _FLEN), jnp.float32),
        scratch_shapes=[pltpu.VMEM((ch, _WPAD * _WPAD + 2), jnp.float32),
                        pltpu.VMEM((9 * ch, _FLEN), jnp.float32)],
    )(xw, f4_raw,
      w1[:, :C], w1[:, C:], col(g_cv1), col(b_cv1),
      w2[:, :C], w2[:, C:], col(g_cv2), col(b_cv2),
      w_m1[:, :, 0, 0], col(g_m1), col(b_m1),
      wm2s, col(g_m2), col(b_m2),
      w3[:, :c_], w3[:, c_:], col(g_cv3), col(b_cv3))

    return out_flat.reshape(c2, 56, _WPAD)[None, :, :, :56]


# speedup vs baseline: 252.2002x; 1.0271x over previous
import functools
import jax
import jax.numpy as jnp
from jax.experimental import pallas as pl
from jax.experimental.pallas import tpu as pltpu

P = 3
L = 8
_DIRS = [(0, 1), (1, 0), (1, 1), (1, -1)]
_PAIRS = []
for _dr, _dc in _DIRS:
    for _i in range(P):
        for _j in range(P):
            if 0 <= _i + _dr < P and 0 <= _j + _dc < P:
                _PAIRS.append((_i * 3 + _j, (_i + _dr) * 3 + (_j + _dc)))

_WPAD = 58
_FLEN = 56 * _WPAD
_XLEN = 3368


def _fused_kernel(nsteps, xb_ref, xfull_ref,
                  w1x_ref, w1f_ref, g1_ref, b1_ref,
                  w2x_ref, w2f_ref, g2_ref, b2_ref,
                  wm1_ref, gm1_ref, bm1_ref,
                  wm2_ref, gm2_ref, bm2_ref,
                  w3a_ref, w3b_ref, g3_ref, b3_ref,
                  out_ref, nb_ref, f4_ref, pad_ref, col_ref):
    i = pl.program_id(0)

    @pl.when(i < nsteps)
    def _glcm():
        cb = xb_ref.shape[0]
        q = jnp.clip(xb_ref[...] * (L - 1), 0, L - 1).astype(jnp.int32)
        for k in range(9):
            o = (k // 3) * _WPAD + (k % 3)
            nb_ref[k * cb:(k + 1) * cb, :] = q[:, o:o + _FLEN]
        nbr = [nb_ref[k * cb:(k + 1) * cb, :] for k in range(9)]
        s8 = [n * L for n in nbr]
        bins = [s8[a] + nbr[b] for a, b in _PAIRS]
        d = [nbr[a] - nbr[b] for a, b in _PAIRS]

        contrast = d[0] * d[0]
        for dk in d[1:]:
            contrast = contrast + dk * dk

        habs = [jnp.abs(dk).astype(jnp.float32) for dk in d]
        homog = 1.0 / (1.0 + habs[0])
        for hk in habs[1:]:
            homog = homog + 1.0 / (1.0 + hk)

        one = jnp.ones_like(bins[0])
        zero = jnp.zeros_like(bins[0])
        inv20 = 1.0 / 20.0
        energy = None
        prod = None
        for k in range(20):
            mk = one
            for l in range(20):
                if l == k:
                    continue
                mk = mk + jnp.where(bins[k] == bins[l], one, zero)
            energy = mk if energy is None else energy + mk
            p = mk.astype(jnp.float32) * inv20 + 1e-6
            prod = p if prod is None else prod * p
        logsum = jnp.log(prod)

        fb = jnp.stack([
            jnp.sum(contrast.astype(jnp.float32), axis=0),
            jnp.sum(energy.astype(jnp.float32), axis=0),
            jnp.sum(logsum, axis=0),
            jnp.sum(homog, axis=0),
        ], axis=0)

        @pl.when(i == 0)
        def _():
            f4_ref[...] = jnp.zeros_like(f4_ref)

        f4_ref[...] += fb

    @pl.when(i == nsteps)
    def _conv():
        cb = xb_ref.shape[0]
        nch = cb * nsteps

        def bnsilu(y, g_ref, b_ref):
            y = y * (g_ref[...] * (1.0 / jnp.sqrt(1.0 + 0.001))) + b_ref[...]
            return y * jax.nn.sigmoid(y)

        def mm(w_ref, a):
            return jax.lax.dot_general(w_ref[...], a,
                                       (((1,), (0,)), ((), ())),
                                       preferred_element_type=jnp.float32)

        x = xfull_ref[:, 59:59 + _FLEN]

        cN = 1.0 / (20.0 * nch)
        rid = jax.lax.broadcasted_iota(jnp.int32, (4, 1), 0)
        scale = jnp.where(rid == 1, 1.0 / (400.0 * nch),
                          jnp.where(rid == 2, -cN, cN))
        f4 = f4_ref[...] * scale

        y1 = mm(w1x_ref, x) + mm(w1f_ref, f4)
        h1 = bnsilu(y1, g1_ref, b1_ref)
        m1 = bnsilu(mm(wm1_ref, h1), gm1_ref, bm1_ref)

        lane = jax.lax.broadcasted_iota(jnp.int32, (1, _FLEN), 1)
        m1 = jnp.where((lane % _WPAD) < 56, m1, 0.0)

        nm = wm2_ref.shape[1] // 9
        pad_ref[...] = jnp.zeros_like(pad_ref)
        pad_ref[:, (_WPAD + 1):(_WPAD + 1) + _FLEN] = m1
        for g in range(9):
            o = (g // 3) * _WPAD + (g % 3)
            col_ref[g * nm:(g + 1) * nm, :] = pad_ref[:, o:o + _FLEN]
        m2 = bnsilu(mm(wm2_ref, col_ref[...]), gm2_ref, bm2_ref)

        mo = h1 + m2
        h2 = bnsilu(mm(w2x_ref, x) + mm(w2f_ref, f4), g2_ref, b2_ref)
        y3 = mm(w3a_ref, mo) + mm(w3b_ref, h2)
        out_ref[...] = bnsilu(y3, g3_ref, b3_ref)


def kernel(x, w_cv1, g_cv1, b_cv1, w_cv2, g_cv2, b_cv2, w_m1, g_m1, b_m1,
           w_m2, g_m2, b_m2, w_cv3, g_cv3, b_cv3):
    B, C, H, W = x.shape
    assert (B, H, W) == (1, 56, 56)
    x0 = x[0]

    xp = jnp.pad(x0, ((0, 0), (1, 1), (1, 1)), mode='edge')
    xp = jnp.pad(xp.reshape(C, _WPAD * _WPAD),
                 ((0, 0), (0, _XLEN - _WPAD * _WPAD)))

    CB = 8
    nsteps = C // CB
    c_ = w_cv1.shape[0]
    ch = w_m1.shape[0]
    c2 = w_cv3.shape[0]
    w1 = w_cv1[:, :, 0, 0]
    w2 = w_cv2[:, :, 0, 0]
    w3 = w_cv3[:, :, 0, 0]
    wm2s = w_m2.transpose(0, 2, 3, 1).reshape(c_, 9 * ch)
    col = lambda v: v[:, None]

    wspecs = [pl.BlockSpec(s, lambda i: (0,) * len(s))
              for s in [(c_, C), (c_, 4), (c_, 1), (c_, 1),
                        (c_, C), (c_, 4), (c_, 1), (c_, 1),
                        (ch, c_), (ch, 1), (ch, 1),
                        (c_, 9 * ch), (c_, 1), (c_, 1),
                        (c2, c_), (c2, c_), (c2, 1), (c2, 1)]]

    out_flat = pl.pallas_call(
        functools.partial(_fused_kernel, nsteps),
        grid=(nsteps + 1,),
        in_specs=[
            pl.BlockSpec((CB, _XLEN),
                         lambda i: (jnp.minimum(i, nsteps - 1), 0)),
            pl.BlockSpec((C, _XLEN), lambda i: (0, 0)),
        ] + wspecs,
        out_specs=pl.BlockSpec((c2, _FLEN), lambda i: (0, 0)),
        out_shape=jax.ShapeDtypeStruct((c2, _FLEN), jnp.float32),
        scratch_shapes=[pltpu.VMEM((9 * CB, _FLEN), jnp.int32),
                        pltpu.VMEM((4, _FLEN), jnp.float32),
                        pltpu.VMEM((ch, _WPAD * _WPAD + 2), jnp.float32),
                        pltpu.VMEM((9 * ch, _FLEN), jnp.float32)],
        compiler_params=pltpu.CompilerParams(
            dimension_semantics=("arbitrary",)),
    )(xp, xp,
      w1[:, :C], w1[:, C:], col(g_cv1), col(b_cv1),
      w2[:, :C], w2[:, C:], col(g_cv2), col(b_cv2),
      w_m1[:, :, 0, 0], col(g_m1), col(b_m1),
      wm2s, col(g_m2), col(b_m2),
      w3[:, :c_], w3[:, c_:], col(g_cv3), col(b_cv3))

    return out_flat.reshape(c2, 56, _WPAD)[None, :, :, :56]


# speedup vs baseline: 336.9426x; 1.3360x over previous
import functools
import jax
import jax.numpy as jnp
from jax.experimental import pallas as pl
from jax.experimental.pallas import tpu as pltpu

P = 3
L = 8
_DIRS = [(0, 1), (1, 0), (1, 1), (1, -1)]
_PAIRS = []
for _dr, _dc in _DIRS:
    for _i in range(P):
        for _j in range(P):
            if 0 <= _i + _dr < P and 0 <= _j + _dc < P:
                _PAIRS.append((_i * 3 + _j, (_i + _dr) * 3 + (_j + _dc)))

_WPAD = 58
_FLEN = 56 * _WPAD
_XLEN = 3368


def _fused_kernel(nsteps, xb_ref, xfull_ref,
                  w1x_ref, w1f_ref, g1_ref, b1_ref,
                  w2x_ref, w2f_ref, g2_ref, b2_ref,
                  wm1_ref, gm1_ref, bm1_ref,
                  wm2_ref, gm2_ref, bm2_ref,
                  w3a_ref, w3b_ref, g3_ref, b3_ref,
                  out_ref, nb_ref, f4_ref, pad_ref, col_ref):
    i = pl.program_id(0)

    @pl.when(i < nsteps)
    def _glcm():
        cb = xb_ref.shape[0]
        q = jnp.floor(jnp.clip(xb_ref[...] * (L - 1), 0, L - 1)).astype(
            jnp.bfloat16)
        for k in range(9):
            o = (k // 3) * _WPAD + (k % 3)
            nb_ref[k * cb:(k + 1) * cb, :] = q[:, o:o + _FLEN]
        nbr = [nb_ref[k * cb:(k + 1) * cb, :] for k in range(9)]
        s8 = [n * jnp.bfloat16(L) for n in nbr]
        bins = [s8[a] + nbr[b] for a, b in _PAIRS]
        d = [nbr[a] - nbr[b] for a, b in _PAIRS]

        contrast = (d[0] * d[0]).astype(jnp.float32)
        for dk in d[1:]:
            contrast = contrast + (dk * dk).astype(jnp.float32)

        habs = [jnp.abs(dk).astype(jnp.float32) for dk in d]
        homog = 1.0 / (1.0 + habs[0])
        for hk in habs[1:]:
            homog = homog + 1.0 / (1.0 + hk)

        one = jnp.ones_like(bins[0])
        zero = jnp.zeros_like(bins[0])
        inv20 = 1.0 / 20.0
        energy = None
        prod = None
        for k0 in range(0, 20, 4):
            mks = [one, one, one, one]
            for l in range(20):
                bl = bins[l]
                for j in range(4):
                    if l == k0 + j:
                        continue
                    mks[j] = mks[j] + jnp.where(bins[k0 + j] == bl, one,
                                                zero)
            for j in range(4):
                mkf = mks[j].astype(jnp.float32)
                energy = mkf if energy is None else energy + mkf
                p = mkf * inv20 + 1e-6
                prod = p if prod is None else prod * p
        logsum = jnp.log(prod)

        fb = jnp.stack([
            jnp.sum(contrast, axis=0),
            jnp.sum(energy, axis=0),
            jnp.sum(logsum, axis=0),
            jnp.sum(homog, axis=0),
        ], axis=0)

        @pl.when(i == 0)
        def _():
            f4_ref[...] = jnp.zeros_like(f4_ref)

        f4_ref[...] += fb

    @pl.when(i == nsteps)
    def _conv():
        cb = xb_ref.shape[0]
        nch = cb * nsteps

        def bnsilu(y, g_ref, b_ref):
            y = y * (g_ref[...] * (1.0 / jnp.sqrt(1.0 + 0.001))) + b_ref[...]
            return y * jax.nn.sigmoid(y)

        def mm(w_ref, a):
            return jax.lax.dot_general(w_ref[...], a,
                                       (((1,), (0,)), ((), ())),
                                       preferred_element_type=jnp.float32)

        x = xfull_ref[:, 59:59 + _FLEN]

        cN = 1.0 / (20.0 * nch)
        rid = jax.lax.broadcasted_iota(jnp.int32, (4, 1), 0)
        scale = jnp.where(rid == 1, 1.0 / (400.0 * nch),
                          jnp.where(rid == 2, -cN, cN))
        f4 = f4_ref[...] * scale

        y1 = mm(w1x_ref, x) + mm(w1f_ref, f4)
        h1 = bnsilu(y1, g1_ref, b1_ref)
        m1 = bnsilu(mm(wm1_ref, h1), gm1_ref, bm1_ref)

        lane = jax.lax.broadcasted_iota(jnp.int32, (1, _FLEN), 1)
        m1 = jnp.where((lane % _WPAD) < 56, m1, 0.0)

        nm = wm2_ref.shape[1] // 9
        pad_ref[...] = jnp.zeros_like(pad_ref)
        pad_ref[:, (_WPAD + 1):(_WPAD + 1) + _FLEN] = m1
        for g in range(9):
            o = (g // 3) * _WPAD + (g % 3)
            col_ref[g * nm:(g + 1) * nm, :] = pad_ref[:, o:o + _FLEN]
        m2 = bnsilu(mm(wm2_ref, col_ref[...]), gm2_ref, bm2_ref)

        mo = h1 + m2
        h2 = bnsilu(mm(w2x_ref, x) + mm(w2f_ref, f4), g2_ref, b2_ref)
        y3 = mm(w3a_ref, mo) + mm(w3b_ref, h2)
        out_ref[...] = bnsilu(y3, g3_ref, b3_ref)


def kernel(x, w_cv1, g_cv1, b_cv1, w_cv2, g_cv2, b_cv2, w_m1, g_m1, b_m1,
           w_m2, g_m2, b_m2, w_cv3, g_cv3, b_cv3):
    B, C, H, W = x.shape
    assert (B, H, W) == (1, 56, 56)
    x0 = x[0]

    xp = jnp.pad(x0, ((0, 0), (1, 1), (1, 1)), mode='edge')
    xp = jnp.pad(xp.reshape(C, _WPAD * _WPAD),
                 ((0, 0), (0, _XLEN - _WPAD * _WPAD)))

    CB = 8
    nsteps = C // CB
    c_ = w_cv1.shape[0]
    ch = w_m1.shape[0]
    c2 = w_cv3.shape[0]
    w1 = w_cv1[:, :, 0, 0]
    w2 = w_cv2[:, :, 0, 0]
    w3 = w_cv3[:, :, 0, 0]
    wm2s = w_m2.transpose(0, 2, 3, 1).reshape(c_, 9 * ch)
    col = lambda v: v[:, None]

    wspecs = [pl.BlockSpec(s, lambda i: (0,) * len(s))
              for s in [(c_, C), (c_, 4), (c_, 1), (c_, 1),
                        (c_, C), (c_, 4), (c_, 1), (c_, 1),
                        (ch, c_), (ch, 1), (ch, 1),
                        (c_, 9 * ch), (c_, 1), (c_, 1),
                        (c2, c_), (c2, c_), (c2, 1), (c2, 1)]]

    out_flat = pl.pallas_call(
        functools.partial(_fused_kernel, nsteps),
        grid=(nsteps + 1,),
        in_specs=[
            pl.BlockSpec((CB, _XLEN),
                         lambda i: (jnp.minimum(i, nsteps - 1), 0)),
            pl.BlockSpec((C, _XLEN), lambda i: (0, 0)),
        ] + wspecs,
        out_specs=pl.BlockSpec((c2, _FLEN), lambda i: (0, 0)),
        out_shape=jax.ShapeDtypeStruct((c2, _FLEN), jnp.float32),
        scratch_shapes=[pltpu.VMEM((9 * CB, _FLEN), jnp.bfloat16),
                        pltpu.VMEM((4, _FLEN), jnp.float32),
                        pltpu.VMEM((ch, _WPAD * _WPAD + 2), jnp.float32),
                        pltpu.VMEM((9 * ch, _FLEN), jnp.float32)],
        compiler_params=pltpu.CompilerParams(
            dimension_semantics=("arbitrary",)),
    )(xp, xp,
      w1[:, :C], w1[:, C:], col(g_cv1), col(b_cv1),
      w2[:, :C], w2[:, C:], col(g_cv2), col(b_cv2),
      w_m1[:, :, 0, 0], col(g_m1), col(b_m1),
      wm2s, col(g_m2), col(b_m2),
      w3[:, :c_], w3[:, c_:], col(g_cv3), col(b_cv3))

    return out_flat.reshape(c2, 56, _WPAD)[None, :, :, :56]
